# baseline (speedup 1.0000x reference)
import functools

import jax
import jax.numpy as jnp
from jax.experimental import pallas as pl
from jax.experimental.pallas import tpu as pltpu

_KS = 5
_HALF = _KS // 2
_RB = 32
_HALO = 8

_D1S = {0: [1, 2], 1: [-2, -1, 0, 1, 2], 2: [-2, -1, 0, 1, 2]}
_PAIRS = [(d0, d1) for d0 in (0, 1, 2) for d1 in _D1S[d0]]


def _lroll(x, s):
    return jnp.roll(x, s, axis=1) if s else x


def _cbl_body(erA_ref, erB_ref, segA_ref, segB_ref, gtA_ref, gtB_ref,
              out_ref, spl_ref, cpl_ref, ppl_ref, tot_ref, scl_ref):
    j = pl.program_id(1)
    C = erA_ref.shape[1]
    W = erA_ref.shape[3]

    normA = jnp.zeros((_RB, W), jnp.float32)
    normB = jnp.zeros((_HALO, W), jnp.float32)
    for c in range(C):
        a = erA_ref[0, c]
        b = erB_ref[0, c]
        normA = normA + a * a
        normB = normB + b * b

    accs = [jnp.zeros((_RB, W), jnp.bfloat16) for _ in _PAIRS]
    for c in range(C):
        a = erA_ref[0, c]
        b = erB_ref[0, c]
        ab = jnp.concatenate([a, b], axis=0)
        a_bf = a.astype(jnp.bfloat16)
        rows = {0: a_bf,
                1: ab[1:1 + _RB].astype(jnp.bfloat16),
                2: ab[2:2 + _RB].astype(jnp.bfloat16)}
        for d1 in (-2, -1, 0, 1, 2):
            ar = _lroll(a_bf, d1)
            for k, (d0, kd1) in enumerate(_PAIRS):
                if kd1 == d1:
                    accs[k] = accs[k] + ar * rows[d0]
    accs = [acc.astype(jnp.float32) for acc in accs]

    gtAB = jnp.concatenate([gtA_ref[0], gtB_ref[0]], axis=0)
    seg0AB = jnp.concatenate([segA_ref[0, 0], segB_ref[0, 0]], axis=0)
    seg1AB = jnp.concatenate([segA_ref[0, 1], segB_ref[0, 1]], axis=0)
    HT = _RB + _HALO

    row_g = jax.lax.broadcasted_iota(jnp.int32, (HT, W), 0) + j * _RB
    col_g = jax.lax.broadcasted_iota(jnp.int32, (HT, W), 1)
    interior = ((row_g >= _HALF) & (row_g < 128 - _HALF)
                & (col_g >= _HALF) & (col_g < W - _HALF))

    gt_c = jnp.where(gtAB == 255, 0, gtAB)
    s1_c = jnp.where(seg1AB == 255, 0, seg1AB)
    posAB = (gt_c * s1_c) > 0
    keptAB = jnp.where(posAB & interior, 1.0, 0.0).astype(jnp.float32)
    lab0AB = seg0AB.astype(jnp.float32)
    lab1AB = seg1AB.astype(jnp.float32)

    normAB = jnp.concatenate([normA, normB], axis=0)
    invAB = 1.0 / jnp.maximum(jnp.sqrt(normAB), 1e-8)

    invA = invAB[:_RB]
    keptA = keptAB[:_RB]
    lab0A = lab0AB[:_RB]
    lab1A = lab1AB[:_RB]

    contrib = jnp.zeros((_RB, W), jnp.float32)
    k = 0
    for d0 in (0, 1, 2):
        inv_r = invAB[d0:d0 + _RB]
        l0_r = lab0AB[d0:d0 + _RB]
        l1_r = lab1AB[d0:d0 + _RB]
        k_r = keptAB[d0:d0 + _RB]
        for d1 in _D1S[d0]:
            sim = _lroll(accs[k], -d1) * invA * _lroll(inv_r, -d1)
            sl = lab0A * _lroll(l0_r, -d1) + lab1A * _lroll(l1_r, -d1)
            diff = sim - sl
            wk = keptA + _lroll(k_r, -d1)
            contrib = contrib + wk * (diff * diff)
            k += 1

    posA = jnp.where(posAB[:_RB], 1.0, 0.0).astype(jnp.float32)

    def _fold8(x):
        return x.reshape(_RB // 8, 8, x.shape[-1]).sum(axis=0)

    s_new = _fold8(contrib)
    c_new = _fold8(keptA)
    p_new = _fold8(posA)

    i = pl.program_id(0)
    nb = pl.num_programs(0)
    nrb = pl.num_programs(1)
    n_shifts = jnp.float32(_KS * _KS - 1)

    @pl.when(j == 0)
    def _():
        spl_ref[...] = s_new
        cpl_ref[...] = c_new
        ppl_ref[...] = p_new

    @pl.when(j != 0)
    def _():
        spl_ref[...] = spl_ref[...] + s_new
        cpl_ref[...] = cpl_ref[...] + c_new
        ppl_ref[...] = ppl_ref[...] + p_new

    @pl.when(j == nrb - 1)
    def _():
        s_b = jnp.sum(spl_ref[...])
        c_b = jnp.sum(cpl_ref[...])
        p_b = jnp.sum(ppl_ref[...])
        has = p_b >= 1.0
        loss_b = (s_b / c_b) / n_shifts
        prev_t = jnp.where(i == 0, 0.0, tot_ref[0, 0])
        prev_s = jnp.where(i == 0, 0.0, scl_ref[0, 0])
        tot_ref[0, 0] = prev_t + jnp.where(has, loss_b, jnp.float32(0.0))
        scl_ref[0, 0] = prev_s + jnp.where(has, 1.0, 0.0)

    @pl.when((i == nb - 1) & (j == nrb - 1))
    def _():
        t = tot_ref[0, 0]
        sc = scl_ref[0, 0]
        t = jnp.where(sc > 0, t / sc, t)
        t = jnp.where(jnp.isnan(t), jnp.float32(0.0), t)
        out_ref[...] = jnp.full((8, 128), t, jnp.float32)


@functools.partial(jax.jit, static_argnames=())
def kernel(er_input, seg_label, gt_boundary_seg, conv10):
    del conv10
    B, C, H, W = er_input.shape
    nrb = H // _RB
    nh = H // _HALO

    def _halo(i, j):
        return jnp.minimum(j * (_RB // _HALO) + _RB // _HALO, nh - 1)

    out = pl.pallas_call(
        _cbl_body,
        grid=(B, nrb),
        in_specs=[
            pl.BlockSpec((1, C, _RB, W), lambda i, j: (i, 0, j, 0)),
            pl.BlockSpec((1, C, _HALO, W), lambda i, j: (i, 0, _halo(i, j), 0)),
            pl.BlockSpec((1, 2, _RB, W), lambda i, j: (i, 0, j, 0)),
            pl.BlockSpec((1, 2, _HALO, W), lambda i, j: (i, 0, _halo(i, j), 0)),
            pl.BlockSpec((1, _RB, W), lambda i, j: (i, j, 0)),
            pl.BlockSpec((1, _HALO, W), lambda i, j: (i, _halo(i, j), 0)),
        ],
        out_specs=pl.BlockSpec((8, W), lambda i, j: (0, 0)),
        out_shape=jax.ShapeDtypeStruct((8, W), jnp.float32),
        scratch_shapes=[
            pltpu.VMEM((8, W), jnp.float32),
            pltpu.VMEM((8, W), jnp.float32),
            pltpu.VMEM((8, W), jnp.float32),
            pltpu.SMEM((1, 1), jnp.float32),
            pltpu.SMEM((1, 1), jnp.float32),
        ],
    )(er_input, er_input, seg_label, seg_label,
      gt_boundary_seg, gt_boundary_seg)

    return out[0, 0]
